# baseline (speedup 1.0000x reference)
import jax
import jax.numpy as jnp
from jax.experimental import pallas as pl
from jax.experimental.pallas import tpu as pltpu

_N_CATS = 10
_CE_W = 1.0
_QWK_W = 0.5
_ORD_W = 0.3


def _loss_body(x_ref, t_ref, out_ref, acc_ref, *, n_tok, n_blocks):
    i = pl.program_id(0)
    x = x_ref[...].T
    t = t_ref[...]
    blk = x.shape[1]
    tf = t.astype(jnp.float32)

    m = jnp.max(x, axis=0, keepdims=True)
    e = jnp.exp(x - m)
    s = jnp.sum(e, axis=0, keepdims=True)
    lse = m + jnp.log(s)

    cats = jax.lax.broadcasted_iota(jnp.int32, x.shape, 0).astype(jnp.float32)
    x_at_t = jnp.sum(jnp.where(cats == tf, x, 0.0), axis=0, keepdims=True)
    ce = lse - x_at_t

    pred = jnp.sum(cats * e, axis=0, keepdims=True) / s
    pen = jnp.abs(pred - tf)
    comb = ce + _ORD_W * pen

    p = jnp.min(jnp.where(x == m, cats, jnp.float32(_N_CATS)), axis=0,
                keepdims=True)

    maskf = (t > 0).astype(jnp.float32)
    d = tf - p
    rows = jnp.concatenate(
        [comb, maskf, maskf * d * d, maskf * tf, maskf * tf * tf,
         maskf * p, maskf * p * p], axis=0)
    part = jnp.sum(rows.reshape(7, blk // 128, 128), axis=1)

    @pl.when(i == 0)
    def _init():
        acc_ref[...] = jnp.zeros_like(acc_ref)

    acc_ref[...] += part

    @pl.when(i == n_blocks - 1)
    def _fini():
        acc = acc_ref[...]
        acc1 = jnp.sum(acc[0, :])
        n = jnp.sum(acc[1, :])
        sumsq = jnp.sum(acc[2, :])
        s1t = jnp.sum(acc[3, :])
        s2t = jnp.sum(acc[4, :])
        s1p = jnp.sum(acc[5, :])
        s2p = jnp.sum(acc[6, :])

        ce_loss = acc1 / n_tok
        nm = jnp.maximum(n, 1.0)
        inv_w = 1.0 / ((_N_CATS - 1.0) ** 2)
        numer = (n - sumsq * inv_w) / nm
        denom = (n * n - (n * s2t + n * s2p - 2.0 * s1t * s1p) * inv_w) / (nm * nm)
        qwk = jnp.where(denom == 0.0, 0.0,
                        numer / jnp.where(denom == 0.0, 1.0, denom))
        qwk = jnp.where(n == 0.0, 0.0, qwk)
        qwk_loss = jnp.where(n == 0.0, 0.0, 1.0 - qwk)
        total = _CE_W * ce_loss + _QWK_W * qwk_loss
        out_ref[0] = total
        out_ref[1] = ce_loss
        out_ref[2] = qwk_loss


def kernel(logits, targets):
    import functools
    b, s, c = logits.shape
    n_tok = b * s
    blk = 4096
    n_blocks = n_tok // blk
    x = logits.reshape(n_tok, c)
    tr = targets.reshape(1, n_tok)

    out = pl.pallas_call(
        functools.partial(_loss_body, n_tok=n_tok, n_blocks=n_blocks),
        grid=(n_blocks,),
        in_specs=[
            pl.BlockSpec((blk, c), lambda i: (i, 0)),
            pl.BlockSpec((1, blk), lambda i: (0, i)),
        ],
        out_shape=jax.ShapeDtypeStruct((3,), jnp.float32),
        out_specs=pl.BlockSpec(memory_space=pltpu.SMEM),
        scratch_shapes=[pltpu.VMEM((7, 128), jnp.float32)],
    )(x, tr)
    return (out[0], out[1], out[2])


# speedup vs baseline: 1.7470x; 1.7470x over previous
import functools

import jax
import jax.numpy as jnp
from jax.experimental import pallas as pl
from jax.experimental.pallas import tpu as pltpu

_N_CATS = 10
_CE_W = 1.0
_QWK_W = 0.5
_ORD_W = 0.3


def _loss_body(x_ref, t_ref, out_ref, acc_ref, *, n_tok, n_blocks):
    i = pl.program_id(0)
    x = x_ref[...]
    t = t_ref[...]
    blk = x.shape[1]
    tf = t.astype(jnp.float32)

    m = jnp.max(x, axis=0, keepdims=True)
    e = jnp.exp(x - m)
    s = jnp.sum(e, axis=0, keepdims=True)
    lse = m + jnp.log(s)

    cats = jax.lax.broadcasted_iota(jnp.int32, x.shape, 0).astype(jnp.float32)
    x_at_t = jnp.sum(jnp.where(cats == tf, x, 0.0), axis=0, keepdims=True)
    ce = lse - x_at_t

    pred = jnp.sum(cats * e, axis=0, keepdims=True) / s
    pen = jnp.abs(pred - tf)
    comb = ce + _ORD_W * pen

    p = jnp.min(jnp.where(x == m, cats, jnp.float32(_N_CATS)), axis=0,
                keepdims=True)

    maskf = (t > 0).astype(jnp.float32)
    d = tf - p
    rows = jnp.concatenate(
        [comb, maskf, maskf * d * d, maskf * tf, maskf * tf * tf,
         maskf * p, maskf * p * p], axis=0)
    part = jnp.sum(rows.reshape(7, blk // 128, 128), axis=1)

    @pl.when(i == 0)
    def _init():
        acc_ref[...] = part

    @pl.when(i > 0)
    def _accum():
        acc_ref[...] += part

    @pl.when(i == n_blocks - 1)
    def _fini():
        acc = acc_ref[...]
        acc1 = jnp.sum(acc[0, :])
        n = jnp.sum(acc[1, :])
        sumsq = jnp.sum(acc[2, :])
        s1t = jnp.sum(acc[3, :])
        s2t = jnp.sum(acc[4, :])
        s1p = jnp.sum(acc[5, :])
        s2p = jnp.sum(acc[6, :])

        ce_loss = acc1 / n_tok
        nm = jnp.maximum(n, 1.0)
        inv_w = 1.0 / ((_N_CATS - 1.0) ** 2)
        numer = (n - sumsq * inv_w) / nm
        denom = (n * n - (n * s2t + n * s2p - 2.0 * s1t * s1p) * inv_w) / (nm * nm)
        qwk = jnp.where(denom == 0.0, 0.0,
                        numer / jnp.where(denom == 0.0, 1.0, denom))
        qwk = jnp.where(n == 0.0, 0.0, qwk)
        qwk_loss = jnp.where(n == 0.0, 0.0, 1.0 - qwk)
        total = _CE_W * ce_loss + _QWK_W * qwk_loss
        out_ref[0] = total
        out_ref[1] = ce_loss
        out_ref[2] = qwk_loss


def kernel(logits, targets):
    b, s, c = logits.shape
    n_tok = b * s
    blk = 4096
    n_blocks = n_tok // blk
    xt = logits.reshape(n_tok, c).T
    tr = targets.reshape(1, n_tok)

    out = pl.pallas_call(
        functools.partial(_loss_body, n_tok=n_tok, n_blocks=n_blocks),
        grid=(n_blocks,),
        in_specs=[
            pl.BlockSpec((c, blk), lambda i: (0, i)),
            pl.BlockSpec((1, blk), lambda i: (0, i)),
        ],
        out_shape=jax.ShapeDtypeStruct((3,), jnp.float32),
        out_specs=pl.BlockSpec(memory_space=pltpu.SMEM),
        scratch_shapes=[pltpu.VMEM((7, 128), jnp.float32)],
    )(xt, tr)
    return (out[0], out[1], out[2])
